# Initial kernel scaffold; baseline (speedup 1.0000x reference)
import jax
import jax.numpy as jnp
from jax.experimental import pallas as pl


def kernel(pl_name_tokens, pl_collaborative, pl_duration_bucket, num_pl_songs_bucket, num_pl_artists_bucket, num_pl_albums_bucket, track_uri_ids, track_name_tokens, artist_uri_ids, artist_name_tokens, album_uri_ids, album_name_tokens, artist_genres_tokens, duration_ms_songs_bucket, track_pop_bucket, artist_pop_bucket, artists_followers_bucket, track_danceability_bucket, track_energy_bucket, track_key, track_loudness_bucket, t_pl_name, t_collab, t_pl_dur, t_num_songs, t_num_artists, t_num_albums, t_track_uri, t_track_name, t_artist_uri, t_artist_name, t_album_uri, t_album_name, t_genres, t_dur_songs, t_track_pop, t_artist_pop, t_followers, t_dance, t_energy, t_key, t_loud, W1, b1, W2, b2):
    raise NotImplementedError("write your pallas kernel here")


# speedup vs baseline: 5.3209x; 5.3209x over previous
import functools

import jax
import jax.numpy as jnp
from jax import lax
from jax.experimental import pallas as pl
from jax.experimental.pallas import tpu as pltpu
from jax.experimental.pallas import tpu_sc as plsc

B = 4096
D = 32
L_SEQ = 50
L_NAME = 16

NC = 2
NS = 16
NW = NC * NS
BPW = B // NW
CHUNK = 16
NCHUNK = BPW // CHUNK

GATHER_LS = [L_NAME, L_SEQ, L_SEQ, L_SEQ, L_SEQ, L_SEQ, L_SEQ, L_SEQ]


def _sc_body(*refs):
  tables = refs[0:8]
  idxs = refs[8:16]
  out = refs[16]
  idx16_v, idx50_v, rows16_v, rows50_v, out_v, sem = refs[17:]

  wid = lax.axis_index("s") * NC + lax.axis_index("c")
  base = wid * BPW

  for f in range(8):
    Lf = GATHER_LS[f]
    idx_v = idx16_v if Lf == L_NAME else idx50_v
    rows_v = rows16_v if Lf == L_NAME else rows50_v
    table = tables[f]
    idx = idxs[f]
    inv_l = 1.0 / Lf
    nrows = CHUNK * Lf
    gsize = 128 if Lf == L_NAME else 80

    def chunk_body(c, _, idx=idx, table=table, idx_v=idx_v, rows_v=rows_v,
                   Lf=Lf, inv_l=inv_l, f=f, nrows=nrows, gsize=gsize):
      b0 = base + c * CHUNK
      pltpu.sync_copy(idx.at[pl.ds(b0 * Lf, nrows)], idx_v)
      copies = [
          pltpu.async_copy(table.at[idx_v.at[pl.ds(g * gsize, gsize)]],
                           rows_v.at[pl.ds(g * gsize, gsize)], sem)
          for g in range(nrows // gsize)
      ]
      for cp in copies:
        cp.wait()

      def item_body(i, _):
        def row_body(j, acc):
          a0, a1 = acc
          r = i * Lf + j
          return (a0 + rows_v[r, pl.ds(0, 16)],
                  a1 + rows_v[r, pl.ds(16, 16)])
        zero = jnp.zeros((16,), jnp.float32)
        a0, a1 = lax.fori_loop(0, Lf, row_body, (zero, zero))
        out_v[i, pl.ds(0, 16)] = a0 * inv_l
        out_v[i, pl.ds(16, 16)] = a1 * inv_l
        return 0

      lax.fori_loop(0, CHUNK, item_body, 0)
      pltpu.sync_copy(out_v, out.at[f, pl.ds(b0, CHUNK)])
      return 0

    lax.fori_loop(0, NCHUNK, chunk_body, 0)


def _sc_gather_pool(tables, idxs):
  mesh = plsc.VectorSubcoreMesh(core_axis_name="c", subcore_axis_name="s",
                                num_cores=NC, num_subcores=NS)
  kern = pl.kernel(
      _sc_body,
      out_type=jax.ShapeDtypeStruct((8, B, D), jnp.float32),
      mesh=mesh,
      compiler_params=pltpu.CompilerParams(use_tc_tiling_on_sc=False),
      scratch_types=[
          pltpu.VMEM((CHUNK * L_NAME,), jnp.int32),
          pltpu.VMEM((CHUNK * L_SEQ,), jnp.int32),
          pltpu.VMEM((CHUNK * L_NAME, D), jnp.float32),
          pltpu.VMEM((CHUNK * L_SEQ, D), jnp.float32),
          pltpu.VMEM((CHUNK, D), jnp.float32),
          pltpu.SemaphoreType.DMA,
      ],
  )
  return kern(*tables, *idxs)


SCALAR_VOCABS = [4, 21, 21, 21, 21]
SEQ_SMALL_VOCABS = [21, 21, 21, 21, 21, 21, 13, 21]

BB = 512


def _tc_body(pooled_ref, sidx_ref, *rest):
  seq_idx_refs = rest[0:8]
  stab_refs = rest[8:13]
  qtab_refs = rest[13:21]
  w1_ref, b1_ref, w2_ref, b2_ref, out_ref = rest[21:]

  parts = []
  parts.append(pooled_ref[0])

  sidx = sidx_ref[...]
  for s, vocab in enumerate(SCALAR_VOCABS):
    ids = sidx[:, s][:, None]
    io = lax.broadcasted_iota(jnp.int32, (BB, vocab), 1)
    oh = (ids == io).astype(jnp.float32)
    parts.append(jnp.dot(oh, stab_refs[s][...],
                         preferred_element_type=jnp.float32))

  for f in range(1, 8):
    parts.append(pooled_ref[f])

  for s, vocab in enumerate(SEQ_SMALL_VOCABS):
    ids = seq_idx_refs[s][...][:, :, None]
    io = lax.broadcasted_iota(jnp.int32, (BB, L_SEQ, vocab), 2)
    cnt = jnp.sum((ids == io).astype(jnp.float32), axis=1)
    parts.append(jnp.dot(cnt, qtab_refs[s][...],
                         preferred_element_type=jnp.float32) * (1.0 / L_SEQ))

  x = jnp.concatenate(parts, axis=1)
  h = jnp.maximum(
      jnp.dot(x, w1_ref[...], preferred_element_type=jnp.float32)
      + b1_ref[...], 0.0)
  out_ref[...] = (jnp.dot(h, w2_ref[...], preferred_element_type=jnp.float32)
                  + b2_ref[...])


def _tc_mlp(pooled, sidx, seq_idxs, stabs, qtabs, W1, b1, W2, b2):
  nfd = W1.shape[0]
  grid = (B // BB,)
  full = lambda shape: pl.BlockSpec(shape, lambda i: (0,) * len(shape))
  in_specs = (
      [pl.BlockSpec((8, BB, D), lambda i: (0, i, 0))] +
      [pl.BlockSpec((BB, 8), lambda i: (i, 0))] +
      [pl.BlockSpec((BB, L_SEQ), lambda i: (i, 0))] * 8 +
      [full(t.shape) for t in stabs] +
      [full(t.shape) for t in qtabs] +
      [full((nfd, 256)), full((1, 256)), full((256, 128)), full((1, 128))]
  )
  return pl.pallas_call(
      _tc_body,
      grid=grid,
      in_specs=in_specs,
      out_specs=pl.BlockSpec((BB, 128), lambda i: (i, 0)),
      out_shape=jax.ShapeDtypeStruct((B, 128), jnp.float32),
  )(pooled, sidx, *seq_idxs, *stabs, *qtabs, W1, b1, W2, b2)


def kernel(pl_name_tokens, pl_collaborative, pl_duration_bucket,
           num_pl_songs_bucket, num_pl_artists_bucket, num_pl_albums_bucket,
           track_uri_ids, track_name_tokens, artist_uri_ids,
           artist_name_tokens, album_uri_ids, album_name_tokens,
           artist_genres_tokens, duration_ms_songs_bucket, track_pop_bucket,
           artist_pop_bucket, artists_followers_bucket,
           track_danceability_bucket, track_energy_bucket, track_key,
           track_loudness_bucket, t_pl_name, t_collab, t_pl_dur, t_num_songs,
           t_num_artists, t_num_albums, t_track_uri, t_track_name,
           t_artist_uri, t_artist_name, t_album_uri, t_album_name, t_genres,
           t_dur_songs, t_track_pop, t_artist_pop, t_followers, t_dance,
           t_energy, t_key, t_loud, W1, b1, W2, b2):
  gather_tables = (t_pl_name, t_track_uri, t_track_name, t_artist_uri,
                   t_artist_name, t_album_uri, t_album_name, t_genres)
  gather_idxs = tuple(
      a.reshape(-1) for a in
      (pl_name_tokens, track_uri_ids, track_name_tokens, artist_uri_ids,
       artist_name_tokens, album_uri_ids, album_name_tokens,
       artist_genres_tokens))
  pooled = _sc_gather_pool(gather_tables, gather_idxs)

  sidx = jnp.stack([pl_collaborative, pl_duration_bucket,
                    num_pl_songs_bucket, num_pl_artists_bucket,
                    num_pl_albums_bucket], axis=1)
  sidx = jnp.pad(sidx, ((0, 0), (0, 3)))
  seq_idxs = (duration_ms_songs_bucket, track_pop_bucket, artist_pop_bucket,
              artists_followers_bucket, track_danceability_bucket,
              track_energy_bucket, track_key, track_loudness_bucket)
  stabs = (t_collab, t_pl_dur, t_num_songs, t_num_artists, t_num_albums)
  qtabs = (t_dur_songs, t_track_pop, t_artist_pop, t_followers, t_dance,
           t_energy, t_key, t_loud)
  return _tc_mlp(pooled, sidx, seq_idxs, stabs, qtabs,
                 W1, b1.reshape(1, 256), W2, b2.reshape(1, 128))


# speedup vs baseline: 5.8945x; 1.1078x over previous
import functools

import jax
import jax.numpy as jnp
from jax import lax
from jax.experimental import pallas as pl
from jax.experimental.pallas import tpu as pltpu
from jax.experimental.pallas import tpu_sc as plsc

B = 4096
D = 32
L_SEQ = 50
L_NAME = 16

NC = 2
NS = 16
NW = NC * NS
BPW = B // NW
CHUNK = 16
NCHUNK = BPW // CHUNK

GATHER_LS = [L_NAME, L_SEQ, L_SEQ, L_SEQ, L_SEQ, L_SEQ, L_SEQ, L_SEQ]


def _sc_body(*refs):
  tables = refs[0:8]
  idxs = refs[8:16]
  out = refs[16]
  (idx_b0, idx_b1, rows_b0, rows_b1, out_v,
   sem_g0, sem_g1, sem_i0, sem_i1, sem_o) = refs[17:]
  idx_bufs = (idx_b0, idx_b1)
  rows_bufs = (rows_b0, rows_b1)
  gsems = (sem_g0, sem_g1)
  isems = (sem_i0, sem_i1)

  wid = lax.axis_index("s") * NC + lax.axis_index("c")
  base = wid * BPW

  def fire_idx(f):
    Lf = GATHER_LS[f]
    buf = idx_bufs[f % 2]
    return pltpu.async_copy(
        idxs[f].at[pl.ds(base * Lf, BPW * Lf)],
        buf.at[pl.ds(0, BPW * Lf)], isems[f % 2])

  def pool(f, c, par):
    Lf = GATHER_LS[f]
    rbuf = rows_bufs[par]
    inv_l = 1.0 / Lf

    def item_body(i, _):
      r0 = i * Lf

      def j_body(j, acc):
        a0, a1, b0, b1 = acc
        r = r0 + 2 * j
        return (a0 + rbuf[r, pl.ds(0, 16)],
                a1 + rbuf[r, pl.ds(16, 16)],
                b0 + rbuf[r + 1, pl.ds(0, 16)],
                b1 + rbuf[r + 1, pl.ds(16, 16)])

      zero = jnp.zeros((16,), jnp.float32)
      a0, a1, b0, b1 = lax.fori_loop(0, Lf // 2, j_body,
                                     (zero, zero, zero, zero), unroll=5)
      out_v[f, c * CHUNK + i, pl.ds(0, 16)] = (a0 + b0) * inv_l
      out_v[f, c * CHUNK + i, pl.ds(16, 16)] = (a1 + b1) * inv_l
      return 0

    lax.fori_loop(0, CHUNK, item_body, 0)

  def gather_copies(f, c, par, make_only):
    Lf = GATHER_LS[f]
    nrows = CHUNK * Lf
    gsize = 128 if Lf == L_NAME else 80
    ibuf = idx_bufs[f % 2]
    rbuf = rows_bufs[par]
    sem = gsems[par]
    mk = pltpu.make_async_copy if make_only else pltpu.async_copy
    return [
        mk(tables[f].at[ibuf.at[pl.ds(c * nrows + k * gsize, gsize)]],
           rbuf.at[pl.ds(k * gsize, gsize)], sem)
        for k in range(nrows // gsize)
    ]

  fire_idx(0).wait()
  for f in range(8):
    gather_copies(f, 0, 0, False)
    if f + 1 < 8:
      idx_pending = fire_idx(f + 1)

    def ring_body(it, _, f=f):
      c0 = it * 2
      gather_copies(f, c0 + 1, 1, False)
      for cp in gather_copies(f, c0, 0, True):
        cp.wait()
      pool(f, c0, 0)

      @pl.when(c0 + 2 < NCHUNK)
      def _():
        gather_copies(f, c0 + 2, 0, False)
      for cp in gather_copies(f, c0 + 1, 1, True):
        cp.wait()
      pool(f, c0 + 1, 1)
      return 0

    lax.fori_loop(0, NCHUNK // 2, ring_body, 0)
    if f + 1 < 8:
      idx_pending.wait()

  outs = [
      pltpu.async_copy(out_v.at[f], out.at[f, pl.ds(base, BPW)], sem_o)
      for f in range(8)
  ]
  for cp in outs:
    cp.wait()


def _sc_gather_pool(tables, idxs):
  mesh = plsc.VectorSubcoreMesh(core_axis_name="c", subcore_axis_name="s",
                                num_cores=NC, num_subcores=NS)
  kern = pl.kernel(
      _sc_body,
      out_type=jax.ShapeDtypeStruct((8, B, D), jnp.float32),
      mesh=mesh,
      compiler_params=pltpu.CompilerParams(use_tc_tiling_on_sc=False),
      scratch_types=[
          pltpu.VMEM((BPW * L_SEQ,), jnp.int32),
          pltpu.VMEM((BPW * L_SEQ,), jnp.int32),
          pltpu.VMEM((CHUNK * L_SEQ, D), jnp.float32),
          pltpu.VMEM((CHUNK * L_SEQ, D), jnp.float32),
          pltpu.VMEM((8, BPW, D), jnp.float32),
          pltpu.SemaphoreType.DMA,
          pltpu.SemaphoreType.DMA,
          pltpu.SemaphoreType.DMA,
          pltpu.SemaphoreType.DMA,
          pltpu.SemaphoreType.DMA,
      ],
  )
  return kern(*tables, *idxs)


SCALAR_VOCABS = [4, 21, 21, 21, 21]
SEQ_SMALL_VOCABS = [21, 21, 21, 21, 21, 21, 13, 21]

BB = 512


def _tc_body(pooled_ref, sidx_ref, *rest):
  seq_idx_refs = rest[0:8]
  stab_refs = rest[8:13]
  qtab_refs = rest[13:21]
  w1_ref, b1_ref, w2_ref, b2_ref, out_ref = rest[21:]

  parts = []
  parts.append(pooled_ref[0])

  sidx = sidx_ref[...]
  for s, vocab in enumerate(SCALAR_VOCABS):
    ids = sidx[:, s][:, None]
    io = lax.broadcasted_iota(jnp.int32, (BB, vocab), 1)
    oh = (ids == io).astype(jnp.float32)
    parts.append(jnp.dot(oh, stab_refs[s][...],
                         preferred_element_type=jnp.float32))

  for f in range(1, 8):
    parts.append(pooled_ref[f])

  for s, vocab in enumerate(SEQ_SMALL_VOCABS):
    ids = seq_idx_refs[s][...][:, :, None]
    io = lax.broadcasted_iota(jnp.int32, (BB, L_SEQ, vocab), 2)
    cnt = jnp.sum((ids == io).astype(jnp.float32), axis=1)
    parts.append(jnp.dot(cnt, qtab_refs[s][...],
                         preferred_element_type=jnp.float32) * (1.0 / L_SEQ))

  x = jnp.concatenate(parts, axis=1)
  h = jnp.maximum(
      jnp.dot(x, w1_ref[...], preferred_element_type=jnp.float32)
      + b1_ref[...], 0.0)
  out_ref[...] = (jnp.dot(h, w2_ref[...], preferred_element_type=jnp.float32)
                  + b2_ref[...])


def _tc_mlp(pooled, sidx, seq_idxs, stabs, qtabs, W1, b1, W2, b2):
  nfd = W1.shape[0]
  grid = (B // BB,)
  full = lambda shape: pl.BlockSpec(shape, lambda i: (0,) * len(shape))
  in_specs = (
      [pl.BlockSpec((8, BB, D), lambda i: (0, i, 0))] +
      [pl.BlockSpec((BB, 8), lambda i: (i, 0))] +
      [pl.BlockSpec((BB, L_SEQ), lambda i: (i, 0))] * 8 +
      [full(t.shape) for t in stabs] +
      [full(t.shape) for t in qtabs] +
      [full((nfd, 256)), full((1, 256)), full((256, 128)), full((1, 128))]
  )
  return pl.pallas_call(
      _tc_body,
      grid=grid,
      in_specs=in_specs,
      out_specs=pl.BlockSpec((BB, 128), lambda i: (i, 0)),
      out_shape=jax.ShapeDtypeStruct((B, 128), jnp.float32),
  )(pooled, sidx, *seq_idxs, *stabs, *qtabs, W1, b1, W2, b2)


def kernel(pl_name_tokens, pl_collaborative, pl_duration_bucket,
           num_pl_songs_bucket, num_pl_artists_bucket, num_pl_albums_bucket,
           track_uri_ids, track_name_tokens, artist_uri_ids,
           artist_name_tokens, album_uri_ids, album_name_tokens,
           artist_genres_tokens, duration_ms_songs_bucket, track_pop_bucket,
           artist_pop_bucket, artists_followers_bucket,
           track_danceability_bucket, track_energy_bucket, track_key,
           track_loudness_bucket, t_pl_name, t_collab, t_pl_dur, t_num_songs,
           t_num_artists, t_num_albums, t_track_uri, t_track_name,
           t_artist_uri, t_artist_name, t_album_uri, t_album_name, t_genres,
           t_dur_songs, t_track_pop, t_artist_pop, t_followers, t_dance,
           t_energy, t_key, t_loud, W1, b1, W2, b2):
  gather_tables = (t_pl_name, t_track_uri, t_track_name, t_artist_uri,
                   t_artist_name, t_album_uri, t_album_name, t_genres)
  gather_idxs = tuple(
      a.reshape(-1) for a in
      (pl_name_tokens, track_uri_ids, track_name_tokens, artist_uri_ids,
       artist_name_tokens, album_uri_ids, album_name_tokens,
       artist_genres_tokens))
  pooled = _sc_gather_pool(gather_tables, gather_idxs)

  sidx = jnp.stack([pl_collaborative, pl_duration_bucket,
                    num_pl_songs_bucket, num_pl_artists_bucket,
                    num_pl_albums_bucket], axis=1)
  sidx = jnp.pad(sidx, ((0, 0), (0, 3)))
  seq_idxs = (duration_ms_songs_bucket, track_pop_bucket, artist_pop_bucket,
              artists_followers_bucket, track_danceability_bucket,
              track_energy_bucket, track_key, track_loudness_bucket)
  stabs = (t_collab, t_pl_dur, t_num_songs, t_num_artists, t_num_albums)
  qtabs = (t_dur_songs, t_track_pop, t_artist_pop, t_followers, t_dance,
           t_energy, t_key, t_loud)
  return _tc_mlp(pooled, sidx, seq_idxs, stabs, qtabs,
                 W1, b1.reshape(1, 256), W2, b2.reshape(1, 128))


# speedup vs baseline: 6.1449x; 1.0425x over previous
import jax
import jax.numpy as jnp
from jax import lax
from jax.experimental import pallas as pl
from jax.experimental.pallas import tpu as pltpu
from jax.experimental.pallas import tpu_sc as plsc

B = 4096
D = 32
L_SEQ = 50
L_NAME = 16
MAX_TOKENS = 50000

NC = 2
NS = 16
NW = NC * NS
BPW = B // NW
CHUNK = 16
NCHUNK = BPW // CHUNK

GATHER_LS = [L_NAME, L_SEQ, L_SEQ, L_SEQ, L_SEQ, L_SEQ, L_SEQ, L_SEQ]


IDX_OFFS = []
_off = 0
for _L in GATHER_LS:
  IDX_OFFS.append(_off)
  _off += B * _L
IDX_TOTAL = _off


def _sc_body(*refs):
  tables = refs[0:8]
  idx_all = refs[8]
  out = refs[9]
  (idx_b0, idx_b1, rows_b0, rows_b1, out_v,
   sem_g0, sem_g1, sem_i0, sem_i1, sem_o) = refs[10:]
  idx_bufs = (idx_b0, idx_b1)
  rows_bufs = (rows_b0, rows_b1)
  gsems = (sem_g0, sem_g1)
  isems = (sem_i0, sem_i1)

  wid = lax.axis_index("s") * NC + lax.axis_index("c")
  base = wid * BPW

  def fire_idx(f):
    Lf = GATHER_LS[f]
    buf = idx_bufs[f % 2]
    return pltpu.async_copy(
        idx_all.at[pl.ds(IDX_OFFS[f] + base * Lf, BPW * Lf)],
        buf.at[pl.ds(0, BPW * Lf)], isems[f % 2])

  def pool(f, c, par):
    Lf = GATHER_LS[f]
    rbuf = rows_bufs[par]
    inv_l = 1.0 / Lf

    def item_body(i, _):
      r0 = i * Lf

      def j_body(j, acc):
        a0, a1, b0, b1 = acc
        r = r0 + 2 * j
        return (a0 + rbuf[r, pl.ds(0, 16)],
                a1 + rbuf[r, pl.ds(16, 16)],
                b0 + rbuf[r + 1, pl.ds(0, 16)],
                b1 + rbuf[r + 1, pl.ds(16, 16)])

      zero = jnp.zeros((16,), jnp.float32)
      a0, a1, b0, b1 = lax.fori_loop(0, Lf // 2, j_body,
                                     (zero, zero, zero, zero), unroll=5)
      out_v[f, c * CHUNK + i, pl.ds(0, 16)] = (a0 + b0) * inv_l
      out_v[f, c * CHUNK + i, pl.ds(16, 16)] = (a1 + b1) * inv_l
      return 0

    lax.fori_loop(0, CHUNK, item_body, 0)

  def gather_copies(f, c, par, make_only):
    Lf = GATHER_LS[f]
    nrows = CHUNK * Lf
    gsize = 128 if Lf == L_NAME else 80
    ibuf = idx_bufs[f % 2]
    rbuf = rows_bufs[par]
    sem = gsems[par]
    mk = pltpu.make_async_copy if make_only else pltpu.async_copy
    return [
        mk(tables[f].at[ibuf.at[pl.ds(c * nrows + k * gsize, gsize)]],
           rbuf.at[pl.ds(k * gsize, gsize)], sem)
        for k in range(nrows // gsize)
    ]

  fire_idx(0).wait()
  for f in range(8):
    gather_copies(f, 0, 0, False)
    if f + 1 < 8:
      idx_pending = fire_idx(f + 1)

    def ring_body(it, _, f=f):
      c0 = it * 2
      gather_copies(f, c0 + 1, 1, False)
      for cp in gather_copies(f, c0, 0, True):
        cp.wait()
      pool(f, c0, 0)

      @pl.when(c0 + 2 < NCHUNK)
      def _():
        gather_copies(f, c0 + 2, 0, False)
      for cp in gather_copies(f, c0 + 1, 1, True):
        cp.wait()
      pool(f, c0 + 1, 1)
      return 0

    lax.fori_loop(0, NCHUNK // 2, ring_body, 0)
    if f + 1 < 8:
      idx_pending.wait()

  outs = [
      pltpu.async_copy(out_v.at[f], out.at[f, pl.ds(base, BPW)], sem_o)
      for f in range(8)
  ]
  for cp in outs:
    cp.wait()


def _sc_gather_pool(tables, idx_all):
  mesh = plsc.VectorSubcoreMesh(core_axis_name="c", subcore_axis_name="s",
                                num_cores=NC, num_subcores=NS)
  kern = pl.kernel(
      _sc_body,
      out_type=jax.ShapeDtypeStruct((8, B, D), jnp.float32),
      mesh=mesh,
      compiler_params=pltpu.CompilerParams(use_tc_tiling_on_sc=False),
      scratch_types=[
          pltpu.VMEM((BPW * L_SEQ,), jnp.int32),
          pltpu.VMEM((BPW * L_SEQ,), jnp.int32),
          pltpu.VMEM((CHUNK * L_SEQ, D), jnp.float32),
          pltpu.VMEM((CHUNK * L_SEQ, D), jnp.float32),
          pltpu.VMEM((8, BPW, D), jnp.float32),
          pltpu.SemaphoreType.DMA,
          pltpu.SemaphoreType.DMA,
          pltpu.SemaphoreType.DMA,
          pltpu.SemaphoreType.DMA,
          pltpu.SemaphoreType.DMA,
      ],
  )
  return kern(*tables, idx_all)


SCALAR_VOCABS = [4, 21, 21, 21, 21]
SEQ_SMALL_VOCABS = [21, 21, 21, 21, 21, 21, 13, 21]

BB = 512


def _tc_small_body(sidx_ref, *rest):
  seq_idx_refs = rest[0:8]
  stab_refs = rest[8:13]
  qtab_refs = rest[13:21]
  w1s_ref, b1_ref, out_ref = rest[21:]

  parts = []
  sidx = sidx_ref[...]
  for s, vocab in enumerate(SCALAR_VOCABS):
    ids = sidx[:, s][:, None]
    io = lax.broadcasted_iota(jnp.int32, (BB, vocab), 1)
    oh = (ids == io).astype(jnp.float32)
    parts.append(jnp.dot(oh, stab_refs[s][...],
                         preferred_element_type=jnp.float32))

  for s, vocab in enumerate(SEQ_SMALL_VOCABS):
    ids = seq_idx_refs[s][...][:, :, None]
    io = lax.broadcasted_iota(jnp.int32, (BB, L_SEQ, vocab), 2)
    cnt = jnp.sum((ids == io).astype(jnp.float32), axis=1)
    parts.append(jnp.dot(cnt, qtab_refs[s][...],
                         preferred_element_type=jnp.float32) * (1.0 / L_SEQ))

  xs = jnp.concatenate(parts, axis=1)
  out_ref[...] = (jnp.dot(xs, w1s_ref[...], preferred_element_type=jnp.float32)
                  + b1_ref[...])


def _tc_final_body(pooled_ref, y1_ref, w1g_ref, w2_ref, b2_ref, out_ref):
  xg = jnp.concatenate([pooled_ref[f] for f in range(8)], axis=1)
  h = jnp.maximum(
      jnp.dot(xg, w1g_ref[...], preferred_element_type=jnp.float32)
      + y1_ref[...], 0.0)
  out_ref[...] = (jnp.dot(h, w2_ref[...], preferred_element_type=jnp.float32)
                  + b2_ref[...])


def _tc_mlp(pooled, sidx, seq_idxs, stabs, qtabs, W1, b1, W2, b2):
  grid = (B // BB,)
  full = lambda shape: pl.BlockSpec(shape, lambda i: (0,) * len(shape))
  w1g = jnp.concatenate([W1[0:32], W1[192:416]], axis=0)
  w1s = jnp.concatenate([W1[32:192], W1[416:672]], axis=0)

  small_specs = (
      [pl.BlockSpec((BB, 8), lambda i: (i, 0))] +
      [pl.BlockSpec((BB, L_SEQ), lambda i: (i, 0))] * 8 +
      [full(t.shape) for t in stabs] +
      [full(t.shape) for t in qtabs] +
      [full((416, 256)), full((1, 256))]
  )
  y1 = pl.pallas_call(
      _tc_small_body,
      grid=grid,
      in_specs=small_specs,
      out_specs=pl.BlockSpec((BB, 256), lambda i: (i, 0)),
      out_shape=jax.ShapeDtypeStruct((B, 256), jnp.float32),
  )(sidx, *seq_idxs, *stabs, *qtabs, w1s, b1)

  final_specs = [
      pl.BlockSpec((8, BB, D), lambda i: (0, i, 0)),
      pl.BlockSpec((BB, 256), lambda i: (i, 0)),
      full((256, 256)), full((256, 128)), full((1, 128)),
  ]
  return pl.pallas_call(
      _tc_final_body,
      grid=grid,
      in_specs=final_specs,
      out_specs=pl.BlockSpec((BB, 128), lambda i: (i, 0)),
      out_shape=jax.ShapeDtypeStruct((B, 128), jnp.float32),
  )(pooled, y1, w1g, W2, b2)


def kernel(pl_name_tokens, pl_collaborative, pl_duration_bucket,
           num_pl_songs_bucket, num_pl_artists_bucket, num_pl_albums_bucket,
           track_uri_ids, track_name_tokens, artist_uri_ids,
           artist_name_tokens, album_uri_ids, album_name_tokens,
           artist_genres_tokens, duration_ms_songs_bucket, track_pop_bucket,
           artist_pop_bucket, artists_followers_bucket,
           track_danceability_bucket, track_energy_bucket, track_key,
           track_loudness_bucket, t_pl_name, t_collab, t_pl_dur, t_num_songs,
           t_num_artists, t_num_albums, t_track_uri, t_track_name,
           t_artist_uri, t_artist_name, t_album_uri, t_album_name, t_genres,
           t_dur_songs, t_track_pop, t_artist_pop, t_followers, t_dance,
           t_energy, t_key, t_loud, W1, b1, W2, b2):
  gather_tables = (t_pl_name, t_track_uri, t_track_name, t_artist_uri,
                   t_artist_name, t_album_uri, t_album_name, t_genres)
  gather_idxs = (pl_name_tokens, track_uri_ids, track_name_tokens,
                 artist_uri_ids, artist_name_tokens, album_uri_ids,
                 album_name_tokens, artist_genres_tokens)
  idx_all = jnp.concatenate([a.reshape(-1) for a in gather_idxs])
  pooled = _sc_gather_pool(gather_tables, idx_all)

  sidx = jnp.stack([pl_collaborative, pl_duration_bucket,
                    num_pl_songs_bucket, num_pl_artists_bucket,
                    num_pl_albums_bucket], axis=1)
  sidx = jnp.pad(sidx, ((0, 0), (0, 3)))
  seq_idxs = (duration_ms_songs_bucket, track_pop_bucket, artist_pop_bucket,
              artists_followers_bucket, track_danceability_bucket,
              track_energy_bucket, track_key, track_loudness_bucket)
  stabs = (t_collab, t_pl_dur, t_num_songs, t_num_artists, t_num_albums)
  qtabs = (t_dur_songs, t_track_pop, t_artist_pop, t_followers, t_dance,
           t_energy, t_key, t_loud)
  return _tc_mlp(pooled, sidx, seq_idxs, stabs, qtabs,
                 W1, b1.reshape(1, 256), W2, b2.reshape(1, 128))
